# baseline (speedup 1.0000x reference)
import functools

import jax
import jax.numpy as jnp
from jax import lax
from jax.experimental import pallas as pl
from jax.experimental.pallas import tpu as pltpu
from jax.experimental.pallas import tpu_sc as plsc

DIM = 64
B = 4096
L = 200
N = B * L
NC = 2
NS = 16
NW = NC * NS
BB = 128
ITEMS = N // BB
PER_W = ITEMS // NW
PER_W2 = PER_W // 2
NBLK = B // BB

_mesh = plsc.VectorSubcoreMesh(core_axis_name="c", subcore_axis_name="s")


@functools.partial(
    pl.kernel,
    mesh=_mesh,
    out_type=jax.ShapeDtypeStruct((L, DIM // 8, NBLK, 8 * BB), jnp.float32),
    scratch_types=[
        pltpu.VMEM((PER_W * BB,), jnp.int32),
        pltpu.VMEM((BB, DIM), jnp.float32),
        pltpu.VMEM((BB, DIM), jnp.float32),
        pltpu.VMEM((DIM // 8, 8 * BB), jnp.float32),
        pltpu.VMEM((DIM // 8, 8 * BB), jnp.float32),
        pltpu.SemaphoreType.DMA,
        pltpu.SemaphoreType.DMA,
        pltpu.SemaphoreType.DMA,
        pltpu.SemaphoreType.DMA,
    ],
    compiler_params=pltpu.CompilerParams(
        use_tc_tiling_on_sc=False, needs_layout_passes=False),
)
def _emb_lookup(idx_hbm, table_hbm, out_hbm, idx_all, rows0, rows1,
                img0, img1, gsem0, gsem1, osem0, osem1):
    wid = lax.axis_index("s") * NC + lax.axis_index("c")
    base = wid * (PER_W * BB)
    pltpu.sync_copy(
        idx_hbm.at[pl.ds(pl.multiple_of(base, 8), PER_W * BB)], idx_all)

    iota16 = lax.iota(jnp.int32, 16)
    rows = (rows0, rows1)
    img = (img0, img1)
    gsem = (gsem0, gsem1)
    osem = (osem0, osem1)

    def gather_start(j, buf):
        off = pl.multiple_of(j * BB, 8)
        pltpu.async_copy(
            table_hbm.at[idx_all.at[pl.ds(off, BB)]], rows[buf], gsem[buf])

    gather_start(0, 0)

    def transpose_item(buf):
        rbuf = rows[buf]
        ibuf = img[buf]

        def col(d, carry):
            dhi = d >> 3
            doff = (d & 7) * BB
            dvec = jnp.full((16,), 0, jnp.int32) + d
            for k in range(8):
                v = plsc.load_gather(rbuf, [iota16 + (16 * k), dvec])
                ibuf[dhi, pl.ds(doff + 16 * k, 16)] = v
            return carry

        lax.fori_loop(0, DIM, col, 0)

    def half(j2, b, j):
        pltpu.make_async_copy(
            table_hbm.at[idx_all.at[pl.ds(0, BB)]], rows[b], gsem[b]).wait()

        nxt = j + 1
        if b == 0:
            gather_start(nxt, 1)
        else:
            @pl.when(j2 < PER_W2 - 1)
            def _():
                gather_start(nxt, 0)

        @pl.when(j2 > 0)
        def _():
            pltpu.make_async_copy(
                img[b], out_hbm.at[0, :, 0, :], osem[b]).wait()

        transpose_item(b)

        g = wid * PER_W + j
        l = g >> 5
        bblk = g & 31
        pltpu.async_copy(img[b], out_hbm.at[l, :, bblk, :], osem[b])

    def body(j2, carry):
        half(j2, 0, 2 * j2)
        half(j2, 1, 2 * j2 + 1)
        return carry

    lax.fori_loop(0, PER_W2, body, 0)
    pltpu.make_async_copy(img[0], out_hbm.at[0, :, 0, :], osem0).wait()
    pltpu.make_async_copy(img[1], out_hbm.at[0, :, 0, :], osem1).wait()


def kernel(token_ids, table):
    tflat = token_ids.transpose(1, 0).reshape(-1).astype(jnp.int32)
    out4 = _emb_lookup(tflat, table)
    out5 = out4.reshape(L, DIM // 8, NBLK, 8, BB)
    return out5.transpose(2, 4, 0, 1, 3).reshape(B, L, DIM)


# speedup vs baseline: 1.8260x; 1.8260x over previous
import functools

import jax
import jax.numpy as jnp
from jax import lax
from jax.experimental import pallas as pl
from jax.experimental.pallas import tpu as pltpu
from jax.experimental.pallas import tpu_sc as plsc

DIM = 64
B = 4096
L = 200
N = B * L
NC = 2
NS = 16
NW = NC * NS
BB = 128
ITEMS = N // BB
PER_W = ITEMS // NW
PER_W2 = PER_W // 2
NBLK = B // BB

_mesh = plsc.VectorSubcoreMesh(core_axis_name="c", subcore_axis_name="s")


@functools.partial(
    pl.kernel,
    mesh=_mesh,
    out_type=jax.ShapeDtypeStruct((L, DIM // 8, NBLK, 8, BB), jnp.float32),
    scratch_types=[
        pltpu.VMEM((PER_W * BB,), jnp.int32),
        pltpu.VMEM((BB, DIM), jnp.float32),
        pltpu.VMEM((BB, DIM), jnp.float32),
        pltpu.VMEM((DIM // 8, 8, BB + 1), jnp.float32),
        pltpu.VMEM((DIM // 8, 8, BB + 1), jnp.float32),
        pltpu.SemaphoreType.DMA,
        pltpu.SemaphoreType.DMA,
        pltpu.SemaphoreType.DMA,
        pltpu.SemaphoreType.DMA,
    ],
    compiler_params=pltpu.CompilerParams(
        use_tc_tiling_on_sc=False, needs_layout_passes=False),
)
def _emb_lookup(idx_hbm, table_hbm, out_hbm, idx_all, rows0, rows1,
                img0, img1, gsem0, gsem1, osem0, osem1):
    wid = lax.axis_index("s") * NC + lax.axis_index("c")
    base = wid * (PER_W * BB)
    pltpu.sync_copy(
        idx_hbm.at[pl.ds(pl.multiple_of(base, 8), PER_W * BB)], idx_all)

    iota16 = lax.iota(jnp.int32, 16)
    rows = (rows0, rows1)
    img = (img0, img1)
    gsem = (gsem0, gsem1)
    osem = (osem0, osem1)

    def gather_start(j, buf):
        off = pl.multiple_of(j * BB, 8)
        pltpu.async_copy(
            table_hbm.at[idx_all.at[pl.ds(off, BB)]], rows[buf], gsem[buf])

    gather_start(0, 0)

    scat_i0 = [(iota16 + 16 * k) >> 3 for k in range(4)]
    scat_i1 = [(iota16 + 16 * k) & 7 for k in range(4)]
    zeros16 = jnp.full((16,), 0, jnp.int32)

    def transpose_item(buf):
        rbuf = rows[buf]
        ibuf = img[buf]

        def col(bb, carry):
            for u in range(2):
                b = 2 * bb + u
                bvec = zeros16 + b
                for k in range(4):
                    v = rbuf[b, pl.ds(16 * k, 16)]
                    plsc.store_scatter(ibuf, [scat_i0[k], scat_i1[k], bvec], v)
            return carry

        lax.fori_loop(0, BB // 2, col, 0)

    def half(j2, b, j):
        pltpu.make_async_copy(
            table_hbm.at[idx_all.at[pl.ds(0, BB)]], rows[b], gsem[b]).wait()

        nxt = j + 1
        if b == 0:
            gather_start(nxt, 1)
        else:
            @pl.when(j2 < PER_W2 - 1)
            def _():
                gather_start(nxt, 0)

        @pl.when(j2 > 0)
        def _():
            pltpu.make_async_copy(
                img[b].at[:, :, pl.ds(0, BB)], out_hbm.at[0, :, 0, :, :],
                osem[b]).wait()

        transpose_item(b)

        g = wid * PER_W + j
        l = g >> 5
        bblk = g & 31
        pltpu.async_copy(
            img[b].at[:, :, pl.ds(0, BB)], out_hbm.at[l, :, bblk, :, :],
            osem[b])

    def body(j2, carry):
        half(j2, 0, 2 * j2)
        half(j2, 1, 2 * j2 + 1)
        return carry

    lax.fori_loop(0, PER_W2, body, 0)
    pltpu.make_async_copy(
        img[0].at[:, :, pl.ds(0, BB)], out_hbm.at[0, :, 0, :, :], osem0).wait()
    pltpu.make_async_copy(
        img[1].at[:, :, pl.ds(0, BB)], out_hbm.at[0, :, 0, :, :], osem1).wait()


def kernel(token_ids, table):
    tflat = token_ids.transpose(1, 0).reshape(-1).astype(jnp.int32)
    out5 = _emb_lookup(tflat, table)
    return out5.transpose(2, 4, 0, 1, 3).reshape(B, L, DIM)


# speedup vs baseline: 2.1717x; 1.1893x over previous
import functools

import jax
import jax.numpy as jnp
from jax import lax
from jax.experimental import pallas as pl
from jax.experimental.pallas import tpu as pltpu
from jax.experimental.pallas import tpu_sc as plsc

DIM = 64
B = 4096
L = 200
N = B * L
NC = 2
NS = 16
NW = NC * NS
BB = 128
ITEMS = N // BB
PER_W = ITEMS // NW
PER_W2 = PER_W // 2
NBLK = B // BB

_mesh = plsc.VectorSubcoreMesh(core_axis_name="c", subcore_axis_name="s")


@functools.partial(
    pl.kernel,
    mesh=_mesh,
    out_type=jax.ShapeDtypeStruct((L, DIM // 8, NBLK, 8, BB), jnp.float32),
    scratch_types=[
        pltpu.VMEM((PER_W * BB,), jnp.int32),
        pltpu.VMEM((BB, DIM), jnp.float32),
        pltpu.VMEM((BB, DIM), jnp.float32),
        pltpu.VMEM((DIM // 8, 8, BB + 1), jnp.float32),
        pltpu.VMEM((DIM // 8, 8, BB + 1), jnp.float32),
        pltpu.SemaphoreType.DMA,
        pltpu.SemaphoreType.DMA,
        pltpu.SemaphoreType.DMA,
        pltpu.SemaphoreType.DMA,
    ],
    compiler_params=pltpu.CompilerParams(
        use_tc_tiling_on_sc=False, needs_layout_passes=False),
)
def _emb_lookup(idx_hbm, table_hbm, out_hbm, idx_all, rows0, rows1,
                img0, img1, gsem0, gsem1, osem0, osem1):
    wid = lax.axis_index("s") * NC + lax.axis_index("c")
    base = wid * (PER_W * BB)
    pltpu.sync_copy(
        idx_hbm.at[pl.ds(pl.multiple_of(base, 8), PER_W * BB)], idx_all)

    iota16 = lax.iota(jnp.int32, 16)
    rows = (rows0, rows1)
    img = (img0, img1)
    gsem = (gsem0, gsem1)
    osem = (osem0, osem1)

    def gather_start(j, buf):
        off = pl.multiple_of(j * BB, 8)
        pltpu.async_copy(
            table_hbm.at[idx_all.at[pl.ds(off, BB)]], rows[buf], gsem[buf])

    gather_start(0, 0)

    scat_i0 = [(iota16 + 16 * k) >> 3 for k in range(4)]
    scat_i1 = [(iota16 + 16 * k) & 7 for k in range(4)]
    zeros16 = jnp.full((16,), 0, jnp.int32)

    def transpose_item(buf):
        rbuf = rows[buf]
        ibuf = img[buf]

        @plsc.parallel_loop(0, BB, unroll=8)
        def col(b):
            bvec = zeros16 + b
            for k in range(4):
                v = rbuf[b, pl.ds(16 * k, 16)]
                plsc.store_scatter(ibuf, [scat_i0[k], scat_i1[k], bvec], v)

    def half(j2, b, j):
        pltpu.make_async_copy(
            table_hbm.at[idx_all.at[pl.ds(0, BB)]], rows[b], gsem[b]).wait()

        nxt = j + 1
        if b == 0:
            gather_start(nxt, 1)
        else:
            @pl.when(j2 < PER_W2 - 1)
            def _():
                gather_start(nxt, 0)

        @pl.when(j2 > 0)
        def _():
            pltpu.make_async_copy(
                img[b].at[:, :, pl.ds(0, BB)], out_hbm.at[0, :, 0, :, :],
                osem[b]).wait()

        transpose_item(b)

        g = wid * PER_W + j
        l = g >> 5
        bblk = g & 31
        pltpu.async_copy(
            img[b].at[:, :, pl.ds(0, BB)], out_hbm.at[l, :, bblk, :, :],
            osem[b])

    def body(j2, carry):
        half(j2, 0, 2 * j2)
        half(j2, 1, 2 * j2 + 1)
        return carry

    lax.fori_loop(0, PER_W2, body, 0)
    pltpu.make_async_copy(
        img[0].at[:, :, pl.ds(0, BB)], out_hbm.at[0, :, 0, :, :], osem0).wait()
    pltpu.make_async_copy(
        img[1].at[:, :, pl.ds(0, BB)], out_hbm.at[0, :, 0, :, :], osem1).wait()


def kernel(token_ids, table):
    tflat = token_ids.transpose(1, 0).reshape(-1).astype(jnp.int32)
    out5 = _emb_lookup(tflat, table)
    return out5.transpose(2, 4, 0, 1, 3).reshape(B, L, DIM)
